# baseline (speedup 1.0000x reference)
import functools

import jax
import jax.numpy as jnp
from jax import lax
from jax.experimental import pallas as pl
from jax.experimental.pallas import tpu as pltpu
from jax.experimental.pallas import tpu_sc as plsc

N = 10000
DF = 128
NG = 128
EPS = 1e-5
NP = 10240
NSUB = 16
NCORE = 2
NW = NCORE * NSUB
STRIPE = NP // NSUB
CH = 128

_mesh = plsc.VectorSubcoreMesh(core_axis_name="c", subcore_axis_name="s")


def _wid(c, s):
    return s * NCORE + c


def _make_deg_kernel(epc):
    @functools.partial(
        pl.kernel,
        out_type=jax.ShapeDtypeStruct((NCORE, NP), jnp.float32),
        mesh=_mesh,
        scratch_types=[
            pltpu.VMEM_SHARED((NP,), jnp.float32),
            pltpu.VMEM((epc + 1, CH), jnp.int32),
            pltpu.VMEM((CH,), jnp.float32),
        ],
    )
    def deg_kernel(dst_hbm, z1_hbm, deg_hbm, deg_sp, idxd_all, ones_v):
        c = lax.axis_index("c")
        s = lax.axis_index("s")
        w = _wid(c, s)
        one = jnp.ones((16,), jnp.float32)
        for j in range(CH // 16):
            ones_v[pl.ds(j * 16, 16)] = one
        pltpu.sync_copy(dst_hbm.at[w], idxd_all)
        pltpu.sync_copy(z1_hbm, deg_sp.at[pl.ds(s * STRIPE, STRIPE)])
        plsc.subcore_barrier()

        def chunk(k, carry):
            pltpu.sync_copy(ones_v, deg_sp.at[idxd_all.at[k]], add=True)
            return carry

        lax.fori_loop(0, epc, chunk, 0)
        plsc.subcore_barrier()
        pltpu.sync_copy(deg_sp.at[pl.ds(s * STRIPE, STRIPE)],
                        deg_hbm.at[c, pl.ds(s * STRIPE, STRIPE)])

    return deg_kernel


def _make_gcn_kernel(epc):
    @functools.partial(
        pl.kernel,
        out_type=jax.ShapeDtypeStruct((NCORE, NP, DF), jnp.float32),
        mesh=_mesh,
        scratch_types=[
            pltpu.VMEM_SHARED((NP, DF), jnp.float32),
            pltpu.VMEM((CH,), jnp.int32),
            pltpu.VMEM((CH,), jnp.int32),
            pltpu.VMEM((CH,), jnp.int32),
            pltpu.VMEM((CH,), jnp.int32),
            pltpu.VMEM((CH, DF), jnp.float32),
            pltpu.VMEM((CH, DF), jnp.float32),
            pltpu.SemaphoreType.DMA,
            pltpu.SemaphoreType.DMA,
        ],
    )
    def gcn_kernel(src_hbm, dst_hbm, tab_hbm, z2_hbm, acc_hbm,
                   acc_sp, src_0, src_1, dst_0, dst_1, rows_a, rows_b,
                   sem_a, sem_b):
        c = lax.axis_index("c")
        s = lax.axis_index("s")
        w = _wid(c, s)
        srcs = (src_0, src_1)
        dsts = (dst_0, dst_1)
        rows = (rows_a, rows_b)
        sems = (sem_a, sem_b)
        pltpu.sync_copy(z2_hbm, acc_sp.at[pl.ds(s * STRIPE, STRIPE)])
        plsc.subcore_barrier()

        def load_idx(k, p):
            pltpu.sync_copy(src_hbm.at[w, k], srcs[p])
            pltpu.sync_copy(dst_hbm.at[w, k], dsts[p])

        def gather(p):
            return pltpu.async_copy(tab_hbm.at[srcs[p]], rows[p], sems[p])

        def wait(p):
            pltpu.make_async_copy(tab_hbm.at[srcs[p]], rows[p], sems[p]
                                  ).wait()

        load_idx(0, 0)
        gather(0)

        def pair(g, carry):
            k0 = 2 * g
            for p in (0, 1):
                o = 1 - p
                load_idx(k0 + p + 1, o)
                gather(o)
                wait(p)
                pltpu.sync_copy(rows[p], acc_sp.at[dsts[p]], add=True)
            return carry

        lax.fori_loop(0, epc // 2, pair, 0)
        wait(0)
        plsc.subcore_barrier()
        pltpu.sync_copy(acc_sp.at[pl.ds(s * STRIPE, STRIPE)],
                        acc_hbm.at[c, pl.ds(s * STRIPE, STRIPE)])

    return gcn_kernel


def _make_gat_kernel(epc):
    @functools.partial(
        pl.kernel,
        out_type=(
            jax.ShapeDtypeStruct((NCORE, NP, DF), jnp.float32),
            jax.ShapeDtypeStruct((NCORE, NP), jnp.float32),
        ),
        mesh=_mesh,
        scratch_types=[
            pltpu.VMEM_SHARED((NP, DF), jnp.float32),
            pltpu.VMEM_SHARED((NP,), jnp.float32),
            pltpu.VMEM((CH,), jnp.int32),
            pltpu.VMEM((CH,), jnp.int32),
            pltpu.VMEM((CH,), jnp.int32),
            pltpu.VMEM((CH,), jnp.int32),
            pltpu.VMEM((CH, DF), jnp.float32),
            pltpu.VMEM((CH, DF), jnp.float32),
            pltpu.VMEM((CH,), jnp.float32),
            pltpu.VMEM((CH,), jnp.float32),
            pltpu.VMEM((CH + 16,), jnp.float32),
            pltpu.SemaphoreType.DMA,
            pltpu.SemaphoreType.DMA,
            pltpu.SemaphoreType.DMA,
        ],
    )
    def gat_kernel(src_hbm, dst_hbm, tab_hbm, es_hbm, ed_hbm, z2_hbm, z1_hbm,
                   acc_hbm, den_hbm,
                   acc_sp, den_sp, src_0, src_1, dst_0, dst_1,
                   rows_a, rows_b, esg_v, edg_v, ee_v,
                   semr_a, semr_b, seme):
        c = lax.axis_index("c")
        s = lax.axis_index("s")
        w = _wid(c, s)
        srcs = (src_0, src_1)
        dsts = (dst_0, dst_1)
        rows = (rows_a, rows_b)
        semr = (semr_a, semr_b)
        pltpu.sync_copy(z2_hbm, acc_sp.at[pl.ds(s * STRIPE, STRIPE)])
        pltpu.sync_copy(z1_hbm, den_sp.at[pl.ds(s * STRIPE, STRIPE)])
        plsc.subcore_barrier()

        def load_idx(k, p):
            pltpu.sync_copy(src_hbm.at[w, k], srcs[p])
            pltpu.sync_copy(dst_hbm.at[w, k], dsts[p])

        def gather_rows(p):
            return pltpu.async_copy(tab_hbm.at[srcs[p]], rows[p], semr[p])

        def wait_rows(p):
            pltpu.make_async_copy(tab_hbm.at[srcs[p]], rows[p], semr[p]
                                  ).wait()

        def gather_esed(p):
            pltpu.async_copy(es_hbm.at[srcs[p]], esg_v, seme)
            pltpu.async_copy(ed_hbm.at[dsts[p]], edg_v, seme)

        def wait_esed(p):
            pltpu.make_async_copy(es_hbm.at[srcs[p]], esg_v, seme).wait()
            pltpu.make_async_copy(ed_hbm.at[dsts[p]], edg_v, seme).wait()

        load_idx(0, 0)
        gather_rows(0)
        gather_esed(0)

        def pair(g, carry):
            k0 = 2 * g
            for p in (0, 1):
                o = 1 - p
                load_idx(k0 + p + 1, o)
                gather_rows(o)
                wait_esed(p)
                for j in range(CH // 16):
                    e = esg_v[pl.ds(j * 16, 16)] + edg_v[pl.ds(j * 16, 16)]
                    e = jnp.where(e >= 0.0, e, e * jnp.float32(0.2))
                    ee_v[pl.ds(j * 16, 16)] = jnp.exp(e)
                gather_esed(o)
                pltpu.sync_copy(ee_v.at[pl.ds(0, CH)],
                                den_sp.at[dsts[p]], add=True)
                wait_rows(p)

                def scale(i, c2):
                    sv = ee_v[pl.ds(i, 16)][0]
                    for j in range(DF // 16):
                        rows[p][i, pl.ds(j * 16, 16)] = (
                            rows[p][i, pl.ds(j * 16, 16)] * sv)
                    return c2

                lax.fori_loop(0, CH, scale, 0)
                pltpu.sync_copy(rows[p], acc_sp.at[dsts[p]], add=True)
            return carry

        lax.fori_loop(0, epc // 2, pair, 0)
        wait_rows(0)
        wait_esed(0)
        plsc.subcore_barrier()
        pltpu.sync_copy(acc_sp.at[pl.ds(s * STRIPE, STRIPE)],
                        acc_hbm.at[c, pl.ds(s * STRIPE, STRIPE)])
        pltpu.sync_copy(den_sp.at[pl.ds(s * STRIPE, STRIPE)],
                        den_hbm.at[c, pl.ds(s * STRIPE, STRIPE)])

    return gat_kernel


def _bn(h, g, b):
    mu = jnp.mean(h, axis=0, keepdims=True)
    var = jnp.mean((h - mu) * (h - mu), axis=0, keepdims=True)
    return (h - mu) * lax.rsqrt(var + EPS) * g + b


def _tc_pre_body(x_ref, w1_ref, deg_ref, hws_ref, dinv_ref):
    deg = deg_ref[0, :N] + deg_ref[1, :N] + 1.0
    dinv = lax.rsqrt(deg)
    dinv_ref[...] = dinv
    hws_ref[...] = (x_ref[...] @ w1_ref[...]) * dinv[:, None]


_tc_pre = pl.pallas_call(
    _tc_pre_body,
    out_shape=(
        jax.ShapeDtypeStruct((N, DF), jnp.float32),
        jax.ShapeDtypeStruct((N,), jnp.float32),
    ),
)


def _tc_mid_body(acc_ref, hws_ref, dinv_ref, b_ref, g_ref, be_ref,
                 hprev_ref, wn_ref, h_ref, hwsn_ref):
    dinv = dinv_ref[...]
    agg = acc_ref[0, :N, :] + acc_ref[1, :N, :] + hws_ref[...]
    gcn = dinv[:, None] * agg + b_ref[...]
    h = jax.nn.relu(_bn(gcn, g_ref[...], be_ref[...])) + hprev_ref[...]
    h_ref[...] = h
    hwsn_ref[...] = (h @ wn_ref[...]) * dinv[:, None]


_tc_mid = pl.pallas_call(
    _tc_mid_body,
    out_shape=(
        jax.ShapeDtypeStruct((N, DF), jnp.float32),
        jax.ShapeDtypeStruct((N, DF), jnp.float32),
    ),
)


def _tc_gatpre_body(acc_ref, hws_ref, dinv_ref, b_ref, g_ref, be_ref,
                    hprev_ref, wg_ref, as_ref, ad_ref,
                    h_ref, hwg_ref, es_ref, ed_ref):
    dinv = dinv_ref[...]
    agg = acc_ref[0, :N, :] + acc_ref[1, :N, :] + hws_ref[...]
    gcn = dinv[:, None] * agg + b_ref[...]
    h = jax.nn.relu(_bn(gcn, g_ref[...], be_ref[...])) + hprev_ref[...]
    h_ref[...] = h
    hwg = h @ wg_ref[...]
    hwg_ref[...] = hwg
    a2 = jnp.concatenate([as_ref[...][:, None], ad_ref[...][:, None]], axis=1)
    esed = hwg @ a2
    zpad = jnp.zeros((NP - N,), jnp.float32)
    es_ref[...] = jnp.concatenate([esed[:, 0], zpad], axis=0)
    ed_ref[...] = jnp.concatenate([esed[:, 1], zpad], axis=0)


_tc_gatpre = pl.pallas_call(
    _tc_gatpre_body,
    out_shape=(
        jax.ShapeDtypeStruct((N, DF), jnp.float32),
        jax.ShapeDtypeStruct((N, DF), jnp.float32),
        jax.ShapeDtypeStruct((NP,), jnp.float32),
        jax.ShapeDtypeStruct((NP,), jnp.float32),
    ),
)


def _tc_post_body(acc_ref, den_ref, es_ref, ed_ref, hwg_ref, h3_ref,
                  bg_ref, ga_ref, bea_ref, batch_ref,
                  wm1_ref, bm1_ref, wh_ref, bh_ref, wm2_ref, bm2_ref,
                  temp_ref, z_ref):
    es = es_ref[:N]
    ed = ed_ref[:N]
    e_self = es + ed
    e_self = jnp.where(e_self >= 0.0, e_self, e_self * jnp.float32(0.2))
    ee_self = jnp.exp(e_self)
    hwg = hwg_ref[...]
    numer = acc_ref[0, :N, :] + acc_ref[1, :N, :] + ee_self[:, None] * hwg
    den = den_ref[0, :N] + den_ref[1, :N] + ee_self
    gat = numer / den[:, None] + bg_ref[...]
    h = jax.nn.relu(_bn(gat, ga_ref[...], bea_ref[...])) + h3_ref[...]
    onehot = (batch_ref[...][:, None]
              == lax.broadcasted_iota(jnp.int32, (N, NG), 1))
    onehot = onehot.astype(jnp.float32)
    sums = lax.dot_general(onehot, h, (((0,), (0,)), ((), ())))
    cnt = jnp.sum(onehot, axis=0)
    pooled = sums / jnp.maximum(cnt, 1.0)[:, None]
    z = jax.nn.relu(pooled @ wm1_ref[...] + bm1_ref[...])
    z = jax.nn.relu(z @ wh_ref[...] + bh_ref[...])
    z = z @ wm2_ref[...] + bm2_ref[...]
    z_ref[...] = z * jnp.exp(temp_ref[0])


_tc_post = pl.pallas_call(
    _tc_post_body,
    out_shape=jax.ShapeDtypeStruct((NG, 128), jnp.float32),
)


def kernel(x, edge_index, batch, W1, b1, W2, b2, W3, b3, g1, be1, g2, be2,
           g3, be3, Wg, a_s, a_d, bg, ga, bea, Wm1, bm1, Wh, bh, Wm2, bm2,
           temp):
    e = edge_index.shape[1]
    quant = NW * CH * 2
    e_pad = -(-e // quant) * quant
    pad = e_pad - e
    epw = e_pad // NW
    epc = epw // CH
    src_p = jnp.concatenate(
        [edge_index[0], jnp.zeros((pad,), edge_index.dtype)])
    dst_p = jnp.concatenate(
        [edge_index[1], jnp.full((pad,), N, edge_index.dtype)])
    src_p = src_p.reshape(NW, epc, CH)
    dst_p = dst_p.reshape(NW, epc, CH)
    src_p = jnp.concatenate(
        [src_p, jnp.zeros((NW, 1, CH), edge_index.dtype)], axis=1)
    dst_p = jnp.concatenate(
        [dst_p, jnp.full((NW, 1, CH), N, edge_index.dtype)], axis=1)
    z1 = jnp.zeros((STRIPE,), jnp.float32)
    z2 = jnp.zeros((STRIPE, DF), jnp.float32)

    deg_k = _make_deg_kernel(epc)
    gcn_k = _make_gcn_kernel(epc)
    gat_k = _make_gat_kernel(epc)

    deg2 = deg_k(dst_p, z1)
    hws1, dinv = _tc_pre(x, W1, deg2)
    acc1 = gcn_k(src_p, dst_p, hws1, z2)
    h1, hws2 = _tc_mid(acc1, hws1, dinv, b1, g1, be1, x, W2)
    acc2 = gcn_k(src_p, dst_p, hws2, z2)
    h2, hws3 = _tc_mid(acc2, hws2, dinv, b2, g2, be2, h1, W3)
    acc3 = gcn_k(src_p, dst_p, hws3, z2)
    h3, hwg, es_p, ed_p = _tc_gatpre(acc3, hws3, dinv, b3, g3, be3, h2,
                                     Wg, a_s, a_d)
    accg, deng = gat_k(src_p, dst_p, hwg, es_p, ed_p, z2, z1)
    z = _tc_post(accg, deng, es_p, ed_p, hwg, h3, bg, ga, bea, batch,
                 Wm1, bm1, Wh, bh, Wm2, bm2, temp)
    return z


# speedup vs baseline: 1.0295x; 1.0295x over previous
import functools

import jax
import jax.numpy as jnp
from jax import lax
from jax.experimental import pallas as pl
from jax.experimental.pallas import tpu as pltpu
from jax.experimental.pallas import tpu_sc as plsc

N = 10000
DF = 128
NG = 128
EPS = 1e-5
NP = 10240
NSUB = 16
NCORE = 2
NW = NCORE * NSUB
STRIPE = NP // NSUB
CH = 128

_mesh = plsc.VectorSubcoreMesh(core_axis_name="c", subcore_axis_name="s")


def _wid(c, s):
    return s * NCORE + c


def _make_deg_kernel(epc):
    @functools.partial(
        pl.kernel,
        out_type=jax.ShapeDtypeStruct((NCORE, NP), jnp.float32),
        mesh=_mesh,
        scratch_types=[
            pltpu.VMEM_SHARED((NP,), jnp.float32),
            pltpu.VMEM((epc + 1, CH), jnp.int32),
            pltpu.VMEM((CH,), jnp.float32),
        ],
    )
    def deg_kernel(dst_hbm, z1_hbm, deg_hbm, deg_sp, idxd_all, ones_v):
        c = lax.axis_index("c")
        s = lax.axis_index("s")
        w = _wid(c, s)
        one = jnp.ones((16,), jnp.float32)
        for j in range(CH // 16):
            ones_v[pl.ds(j * 16, 16)] = one
        pltpu.sync_copy(dst_hbm.at[w], idxd_all)
        pltpu.sync_copy(z1_hbm, deg_sp.at[pl.ds(s * STRIPE, STRIPE)])
        plsc.subcore_barrier()

        def chunk(k, carry):
            pltpu.sync_copy(ones_v, deg_sp.at[idxd_all.at[k]], add=True)
            return carry

        lax.fori_loop(0, epc, chunk, 0)
        plsc.subcore_barrier()
        pltpu.sync_copy(deg_sp.at[pl.ds(s * STRIPE, STRIPE)],
                        deg_hbm.at[c, pl.ds(s * STRIPE, STRIPE)])

    return deg_kernel


def _make_gcn_kernel(epc):
    @functools.partial(
        pl.kernel,
        out_type=jax.ShapeDtypeStruct((NCORE, NP, DF), jnp.float32),
        mesh=_mesh,
        scratch_types=[
            pltpu.VMEM_SHARED((NP, DF), jnp.float32),
            pltpu.VMEM((CH,), jnp.int32),
            pltpu.VMEM((CH,), jnp.int32),
            pltpu.VMEM((CH, DF), jnp.float32),
            pltpu.SemaphoreType.DMA,
        ],
    )
    def gcn_kernel(src_hbm, dst_hbm, tab_hbm, z2_hbm, acc_hbm,
                   acc_sp, idxs_v, idxd_v, rows_v, sem):
        c = lax.axis_index("c")
        s = lax.axis_index("s")
        w = _wid(c, s)
        pltpu.sync_copy(z2_hbm, acc_sp.at[pl.ds(s * STRIPE, STRIPE)])
        plsc.subcore_barrier()

        def chunk(k, carry):
            pltpu.sync_copy(src_hbm.at[w, k], idxs_v)
            pltpu.sync_copy(dst_hbm.at[w, k], idxd_v)
            pltpu.async_copy(tab_hbm.at[idxs_v], rows_v, sem).wait()
            pltpu.sync_copy(rows_v, acc_sp.at[idxd_v], add=True)
            return carry

        lax.fori_loop(0, epc, chunk, 0)
        plsc.subcore_barrier()
        pltpu.sync_copy(acc_sp.at[pl.ds(s * STRIPE, STRIPE)],
                        acc_hbm.at[c, pl.ds(s * STRIPE, STRIPE)])

    return gcn_kernel


def _make_gat_kernel(epc):
    @functools.partial(
        pl.kernel,
        out_type=(
            jax.ShapeDtypeStruct((NCORE, NP, DF), jnp.float32),
            jax.ShapeDtypeStruct((NCORE, NP), jnp.float32),
        ),
        mesh=_mesh,
        scratch_types=[
            pltpu.VMEM_SHARED((NP, DF), jnp.float32),
            pltpu.VMEM_SHARED((NP,), jnp.float32),
            pltpu.VMEM((CH,), jnp.int32),
            pltpu.VMEM((CH,), jnp.int32),
            pltpu.VMEM((CH, DF), jnp.float32),
            pltpu.VMEM((CH,), jnp.float32),
            pltpu.VMEM((CH,), jnp.float32),
            pltpu.VMEM((CH,), jnp.float32),
            pltpu.SemaphoreType.DMA,
            pltpu.SemaphoreType.DMA,
        ],
    )
    def gat_kernel(src_hbm, dst_hbm, tab_hbm, es_hbm, ed_hbm, z2_hbm, z1_hbm,
                   acc_hbm, den_hbm,
                   acc_sp, den_sp, idxs_v, idxd_v,
                   rows_v, esg_v, edg_v, ee_v, semr, seme):
        c = lax.axis_index("c")
        s = lax.axis_index("s")
        w = _wid(c, s)
        pltpu.sync_copy(z2_hbm, acc_sp.at[pl.ds(s * STRIPE, STRIPE)])
        pltpu.sync_copy(z1_hbm, den_sp.at[pl.ds(s * STRIPE, STRIPE)])
        plsc.subcore_barrier()

        def chunk(k, carry):
            pltpu.sync_copy(src_hbm.at[w, k], idxs_v)
            pltpu.sync_copy(dst_hbm.at[w, k], idxd_v)
            cp = pltpu.async_copy(tab_hbm.at[idxs_v], rows_v, semr)
            cpe = pltpu.async_copy(es_hbm.at[idxs_v], esg_v, seme)
            cpd = pltpu.async_copy(ed_hbm.at[idxd_v], edg_v, seme)
            cpe.wait()
            cpd.wait()
            for j in range(CH // 16):
                e = esg_v[pl.ds(j * 16, 16)] + edg_v[pl.ds(j * 16, 16)]
                e = jnp.where(e >= 0.0, e, e * jnp.float32(0.2))
                ee_v[pl.ds(j * 16, 16)] = jnp.exp(e)
            pltpu.sync_copy(ee_v, den_sp.at[idxd_v], add=True)
            cp.wait()

            def scale(j, c2):
                ee16 = ee_v[pl.ds(j * 16, 16)]
                for l in range(16):
                    sv = ee16[jnp.full((16,), l, jnp.int32)]
                    i = j * 16 + l
                    for q in range(DF // 16):
                        rows_v[i, pl.ds(q * 16, 16)] = (
                            rows_v[i, pl.ds(q * 16, 16)] * sv)
                return c2

            lax.fori_loop(0, CH // 16, scale, 0)
            pltpu.sync_copy(rows_v, acc_sp.at[idxd_v], add=True)
            return carry

        lax.fori_loop(0, epc, chunk, 0)
        plsc.subcore_barrier()
        pltpu.sync_copy(acc_sp.at[pl.ds(s * STRIPE, STRIPE)],
                        acc_hbm.at[c, pl.ds(s * STRIPE, STRIPE)])
        pltpu.sync_copy(den_sp.at[pl.ds(s * STRIPE, STRIPE)],
                        den_hbm.at[c, pl.ds(s * STRIPE, STRIPE)])

    return gat_kernel


def _bn(h, g, b):
    mu = jnp.mean(h, axis=0, keepdims=True)
    var = jnp.mean((h - mu) * (h - mu), axis=0, keepdims=True)
    return (h - mu) * lax.rsqrt(var + EPS) * g + b


def _tc_pre_body(x_ref, w1_ref, deg_ref, hws_ref, dinv_ref):
    deg = deg_ref[0, :N] + deg_ref[1, :N] + 1.0
    dinv = lax.rsqrt(deg)
    dinv_ref[...] = dinv
    hws_ref[...] = (x_ref[...] @ w1_ref[...]) * dinv[:, None]


_tc_pre = pl.pallas_call(
    _tc_pre_body,
    out_shape=(
        jax.ShapeDtypeStruct((N, DF), jnp.float32),
        jax.ShapeDtypeStruct((N,), jnp.float32),
    ),
)


def _tc_mid_body(acc_ref, hws_ref, dinv_ref, b_ref, g_ref, be_ref,
                 hprev_ref, wn_ref, h_ref, hwsn_ref):
    dinv = dinv_ref[...]
    agg = acc_ref[0, :N, :] + acc_ref[1, :N, :] + hws_ref[...]
    gcn = dinv[:, None] * agg + b_ref[...]
    h = jax.nn.relu(_bn(gcn, g_ref[...], be_ref[...])) + hprev_ref[...]
    h_ref[...] = h
    hwsn_ref[...] = (h @ wn_ref[...]) * dinv[:, None]


_tc_mid = pl.pallas_call(
    _tc_mid_body,
    out_shape=(
        jax.ShapeDtypeStruct((N, DF), jnp.float32),
        jax.ShapeDtypeStruct((N, DF), jnp.float32),
    ),
)


def _tc_gatpre_body(acc_ref, hws_ref, dinv_ref, b_ref, g_ref, be_ref,
                    hprev_ref, wg_ref, as_ref, ad_ref,
                    h_ref, hwg_ref, es_ref, ed_ref):
    dinv = dinv_ref[...]
    agg = acc_ref[0, :N, :] + acc_ref[1, :N, :] + hws_ref[...]
    gcn = dinv[:, None] * agg + b_ref[...]
    h = jax.nn.relu(_bn(gcn, g_ref[...], be_ref[...])) + hprev_ref[...]
    h_ref[...] = h
    hwg = h @ wg_ref[...]
    hwg_ref[...] = hwg
    a2 = jnp.concatenate([as_ref[...][:, None], ad_ref[...][:, None]], axis=1)
    esed = hwg @ a2
    zpad = jnp.zeros((NP - N,), jnp.float32)
    es_ref[...] = jnp.concatenate([esed[:, 0], zpad], axis=0)
    ed_ref[...] = jnp.concatenate([esed[:, 1], zpad], axis=0)


_tc_gatpre = pl.pallas_call(
    _tc_gatpre_body,
    out_shape=(
        jax.ShapeDtypeStruct((N, DF), jnp.float32),
        jax.ShapeDtypeStruct((N, DF), jnp.float32),
        jax.ShapeDtypeStruct((NP,), jnp.float32),
        jax.ShapeDtypeStruct((NP,), jnp.float32),
    ),
)


def _tc_post_body(acc_ref, den_ref, es_ref, ed_ref, hwg_ref, h3_ref,
                  bg_ref, ga_ref, bea_ref, batch_ref,
                  wm1_ref, bm1_ref, wh_ref, bh_ref, wm2_ref, bm2_ref,
                  temp_ref, z_ref):
    es = es_ref[:N]
    ed = ed_ref[:N]
    e_self = es + ed
    e_self = jnp.where(e_self >= 0.0, e_self, e_self * jnp.float32(0.2))
    ee_self = jnp.exp(e_self)
    hwg = hwg_ref[...]
    numer = acc_ref[0, :N, :] + acc_ref[1, :N, :] + ee_self[:, None] * hwg
    den = den_ref[0, :N] + den_ref[1, :N] + ee_self
    gat = numer / den[:, None] + bg_ref[...]
    h = jax.nn.relu(_bn(gat, ga_ref[...], bea_ref[...])) + h3_ref[...]
    onehot = (batch_ref[...][:, None]
              == lax.broadcasted_iota(jnp.int32, (N, NG), 1))
    onehot = onehot.astype(jnp.float32)
    sums = lax.dot_general(onehot, h, (((0,), (0,)), ((), ())))
    cnt = jnp.sum(onehot, axis=0)
    pooled = sums / jnp.maximum(cnt, 1.0)[:, None]
    z = jax.nn.relu(pooled @ wm1_ref[...] + bm1_ref[...])
    z = jax.nn.relu(z @ wh_ref[...] + bh_ref[...])
    z = z @ wm2_ref[...] + bm2_ref[...]
    z_ref[...] = z * jnp.exp(temp_ref[0])


_tc_post = pl.pallas_call(
    _tc_post_body,
    out_shape=jax.ShapeDtypeStruct((NG, 128), jnp.float32),
)


def kernel(x, edge_index, batch, W1, b1, W2, b2, W3, b3, g1, be1, g2, be2,
           g3, be3, Wg, a_s, a_d, bg, ga, bea, Wm1, bm1, Wh, bh, Wm2, bm2,
           temp):
    e = edge_index.shape[1]
    quant = NW * CH * 2
    e_pad = -(-e // quant) * quant
    pad = e_pad - e
    epw = e_pad // NW
    epc = epw // CH
    src_p = jnp.concatenate(
        [edge_index[0], jnp.zeros((pad,), edge_index.dtype)])
    dst_p = jnp.concatenate(
        [edge_index[1], jnp.full((pad,), N, edge_index.dtype)])
    src_p = src_p.reshape(NW, epc, CH)
    dst_p = dst_p.reshape(NW, epc, CH)
    src_p = jnp.concatenate(
        [src_p, jnp.zeros((NW, 1, CH), edge_index.dtype)], axis=1)
    dst_p = jnp.concatenate(
        [dst_p, jnp.full((NW, 1, CH), N, edge_index.dtype)], axis=1)
    z1 = jnp.zeros((STRIPE,), jnp.float32)
    z2 = jnp.zeros((STRIPE, DF), jnp.float32)

    deg_k = _make_deg_kernel(epc)
    gcn_k = _make_gcn_kernel(epc)
    gat_k = _make_gat_kernel(epc)

    deg2 = deg_k(dst_p, z1)
    hws1, dinv = _tc_pre(x, W1, deg2)
    acc1 = gcn_k(src_p, dst_p, hws1, z2)
    h1, hws2 = _tc_mid(acc1, hws1, dinv, b1, g1, be1, x, W2)
    acc2 = gcn_k(src_p, dst_p, hws2, z2)
    h2, hws3 = _tc_mid(acc2, hws2, dinv, b2, g2, be2, h1, W3)
    acc3 = gcn_k(src_p, dst_p, hws3, z2)
    h3, hwg, es_p, ed_p = _tc_gatpre(acc3, hws3, dinv, b3, g3, be3, h2,
                                     Wg, a_s, a_d)
    accg, deng = gat_k(src_p, dst_p, hwg, es_p, ed_p, z2, z1)
    z = _tc_post(accg, deng, es_p, ed_p, hwg, h3, bg, ga, bea, batch,
                 Wm1, bm1, Wh, bh, Wm2, bm2, temp)
    return z


# speedup vs baseline: 1.6292x; 1.5824x over previous
import functools

import jax
import jax.numpy as jnp
from jax import lax
from jax.experimental import pallas as pl
from jax.experimental.pallas import tpu as pltpu
from jax.experimental.pallas import tpu_sc as plsc

N = 10000
DF = 128
NG = 128
EPS = 1e-5
NP = 10240
NSUB = 16
NCORE = 2
NW = NCORE * NSUB
STRIPE = NP // NSUB
CH = 128

_mesh = plsc.VectorSubcoreMesh(core_axis_name="c", subcore_axis_name="s")


def _wid(c, s):
    return s * NCORE + c


def _zero_rows(rows_v):
    z = jnp.zeros((16,), jnp.float32)

    def zrow(i, cy):
        for q in range(DF // 16):
            rows_v[i, pl.ds(q * 16, 16)] = z
        return cy

    lax.fori_loop(0, CH, zrow, 0)


def _zero_vec(v):
    z = jnp.zeros((16,), jnp.float32)
    for q in range(CH // 16):
        v[pl.ds(q * 16, 16)] = z


def _make_deg_kernel(epc):
    @functools.partial(
        pl.kernel,
        out_type=jax.ShapeDtypeStruct((NCORE, NP), jnp.float32),
        mesh=_mesh,
        scratch_types=[
            pltpu.VMEM_SHARED((NP,), jnp.float32),
            pltpu.VMEM((epc, CH), jnp.int32),
            pltpu.VMEM((CH,), jnp.float32),
        ],
    )
    def deg_kernel(dst_hbm, deg_hbm, deg_sp, idxd_all, ones_v):
        c = lax.axis_index("c")
        s = lax.axis_index("s")
        w = _wid(c, s)
        _zero_vec(ones_v)
        for r in range(STRIPE // CH):
            pltpu.sync_copy(ones_v,
                            deg_sp.at[pl.ds(s * STRIPE + r * CH, CH)])
        one = jnp.ones((16,), jnp.float32)
        for j in range(CH // 16):
            ones_v[pl.ds(j * 16, 16)] = one
        pltpu.sync_copy(dst_hbm.at[w], idxd_all)
        plsc.subcore_barrier()

        def chunk(k, carry):
            pltpu.sync_copy(ones_v, deg_sp.at[idxd_all.at[k]], add=True)
            return carry

        lax.fori_loop(0, epc, chunk, 0)
        plsc.subcore_barrier()
        pltpu.sync_copy(deg_sp.at[pl.ds(s * STRIPE, STRIPE)],
                        deg_hbm.at[c, pl.ds(s * STRIPE, STRIPE)])

    return deg_kernel


def _make_gcn_kernel(epc):
    epw = epc * CH

    @functools.partial(
        pl.kernel,
        out_type=jax.ShapeDtypeStruct((NCORE, NP, DF), jnp.float32),
        mesh=_mesh,
        scratch_types=[
            pltpu.VMEM_SHARED((NP, DF), jnp.float32),
            pltpu.VMEM((CH,), jnp.int32),
            pltpu.VMEM((CH,), jnp.int32),
            pltpu.VMEM((CH, DF), jnp.float32),
            pltpu.SemaphoreType.DMA,
        ],
    )
    def gcn_kernel(src_hbm, dst_hbm, tab_hbm, acc_hbm,
                   acc_sp, idxs_v, idxd_v, rows_v, sem):
        c = lax.axis_index("c")
        s = lax.axis_index("s")
        _zero_rows(rows_v)
        for r in range(STRIPE // CH):
            pltpu.sync_copy(rows_v,
                            acc_sp.at[pl.ds(s * STRIPE + r * CH, CH)])
        plsc.subcore_barrier()
        base = _wid(c, s) * epw

        def chunk(k, carry):
            pltpu.sync_copy(src_hbm.at[pl.ds(base + k * CH, CH)], idxs_v)
            pltpu.sync_copy(dst_hbm.at[pl.ds(base + k * CH, CH)], idxd_v)
            pltpu.async_copy(tab_hbm.at[idxs_v], rows_v, sem).wait()
            pltpu.sync_copy(rows_v, acc_sp.at[idxd_v], add=True)
            return carry

        lax.fori_loop(0, epc, chunk, 0)
        plsc.subcore_barrier()
        pltpu.sync_copy(acc_sp.at[pl.ds(s * STRIPE, STRIPE)],
                        acc_hbm.at[c, pl.ds(s * STRIPE, STRIPE)])

    return gcn_kernel


def _make_gat_kernel(epc):
    epw = epc * CH

    @functools.partial(
        pl.kernel,
        out_type=(
            jax.ShapeDtypeStruct((NCORE, NP, DF), jnp.float32),
            jax.ShapeDtypeStruct((NCORE, NP), jnp.float32),
        ),
        mesh=_mesh,
        scratch_types=[
            pltpu.VMEM_SHARED((NP, DF), jnp.float32),
            pltpu.VMEM_SHARED((NP,), jnp.float32),
            pltpu.VMEM((CH,), jnp.int32),
            pltpu.VMEM((CH,), jnp.int32),
            pltpu.VMEM((CH, DF), jnp.float32),
            pltpu.VMEM((CH,), jnp.float32),
            pltpu.VMEM((CH,), jnp.float32),
            pltpu.VMEM((CH,), jnp.float32),
            pltpu.SemaphoreType.DMA,
            pltpu.SemaphoreType.DMA,
        ],
    )
    def gat_kernel(src_hbm, dst_hbm, tab_hbm, es_hbm, ed_hbm,
                   acc_hbm, den_hbm,
                   acc_sp, den_sp, idxs_v, idxd_v,
                   rows_v, esg_v, edg_v, ee_v, semr, seme):
        c = lax.axis_index("c")
        s = lax.axis_index("s")
        _zero_rows(rows_v)
        for r in range(STRIPE // CH):
            pltpu.sync_copy(rows_v,
                            acc_sp.at[pl.ds(s * STRIPE + r * CH, CH)])
        _zero_vec(esg_v)
        for r in range(STRIPE // CH):
            pltpu.sync_copy(esg_v,
                            den_sp.at[pl.ds(s * STRIPE + r * CH, CH)])
        plsc.subcore_barrier()
        base = _wid(c, s) * epw

        def chunk(k, carry):
            pltpu.sync_copy(src_hbm.at[pl.ds(base + k * CH, CH)], idxs_v)
            pltpu.sync_copy(dst_hbm.at[pl.ds(base + k * CH, CH)], idxd_v)
            cp = pltpu.async_copy(tab_hbm.at[idxs_v], rows_v, semr)
            cpe = pltpu.async_copy(es_hbm.at[idxs_v], esg_v, seme)
            cpd = pltpu.async_copy(ed_hbm.at[idxd_v], edg_v, seme)
            cpe.wait()
            cpd.wait()
            for j in range(CH // 16):
                e = esg_v[pl.ds(j * 16, 16)] + edg_v[pl.ds(j * 16, 16)]
                e = jnp.where(e >= 0.0, e, e * jnp.float32(0.2))
                ee_v[pl.ds(j * 16, 16)] = jnp.exp(e)
            pltpu.sync_copy(ee_v, den_sp.at[idxd_v], add=True)
            cp.wait()

            def scale(j, c2):
                ee16 = ee_v[pl.ds(j * 16, 16)]
                for l in range(16):
                    sv = ee16[jnp.full((16,), l, jnp.int32)]
                    i = j * 16 + l
                    for q in range(DF // 16):
                        rows_v[i, pl.ds(q * 16, 16)] = (
                            rows_v[i, pl.ds(q * 16, 16)] * sv)
                return c2

            lax.fori_loop(0, CH // 16, scale, 0)
            pltpu.sync_copy(rows_v, acc_sp.at[idxd_v], add=True)
            return carry

        lax.fori_loop(0, epc, chunk, 0)
        plsc.subcore_barrier()
        pltpu.sync_copy(acc_sp.at[pl.ds(s * STRIPE, STRIPE)],
                        acc_hbm.at[c, pl.ds(s * STRIPE, STRIPE)])
        pltpu.sync_copy(den_sp.at[pl.ds(s * STRIPE, STRIPE)],
                        den_hbm.at[c, pl.ds(s * STRIPE, STRIPE)])

    return gat_kernel


def _bn(h, g, b):
    mu = jnp.mean(h, axis=0, keepdims=True)
    var = jnp.mean((h - mu) * (h - mu), axis=0, keepdims=True)
    return (h - mu) * lax.rsqrt(var + EPS) * g + b


def _tc_pre_body(x_ref, w1_ref, deg_ref, hws_ref, dinv_ref):
    deg = deg_ref[0, :N] + deg_ref[1, :N] + 1.0
    dinv = lax.rsqrt(deg)
    dinv_ref[...] = dinv
    hws_ref[...] = (x_ref[...] @ w1_ref[...]) * dinv[:, None]


_tc_pre = pl.pallas_call(
    _tc_pre_body,
    out_shape=(
        jax.ShapeDtypeStruct((N, DF), jnp.float32),
        jax.ShapeDtypeStruct((N,), jnp.float32),
    ),
)


def _tc_mid_body(acc_ref, hws_ref, dinv_ref, b_ref, g_ref, be_ref,
                 hprev_ref, wn_ref, h_ref, hwsn_ref):
    dinv = dinv_ref[...]
    agg = acc_ref[0, :N, :] + acc_ref[1, :N, :] + hws_ref[...]
    gcn = dinv[:, None] * agg + b_ref[...]
    h = jax.nn.relu(_bn(gcn, g_ref[...], be_ref[...])) + hprev_ref[...]
    h_ref[...] = h
    hwsn_ref[...] = (h @ wn_ref[...]) * dinv[:, None]


_tc_mid = pl.pallas_call(
    _tc_mid_body,
    out_shape=(
        jax.ShapeDtypeStruct((N, DF), jnp.float32),
        jax.ShapeDtypeStruct((N, DF), jnp.float32),
    ),
)


def _tc_gatpre_body(acc_ref, hws_ref, dinv_ref, b_ref, g_ref, be_ref,
                    hprev_ref, wg_ref, as_ref, ad_ref,
                    h_ref, hwg_ref, es_ref, ed_ref):
    dinv = dinv_ref[...]
    agg = acc_ref[0, :N, :] + acc_ref[1, :N, :] + hws_ref[...]
    gcn = dinv[:, None] * agg + b_ref[...]
    h = jax.nn.relu(_bn(gcn, g_ref[...], be_ref[...])) + hprev_ref[...]
    h_ref[...] = h
    hwg = h @ wg_ref[...]
    hwg_ref[...] = hwg
    a2 = jnp.concatenate([as_ref[...][:, None], ad_ref[...][:, None]], axis=1)
    esed = hwg @ a2
    zpad = jnp.zeros((NP - N,), jnp.float32)
    es_ref[...] = jnp.concatenate([esed[:, 0], zpad], axis=0)
    ed_ref[...] = jnp.concatenate([esed[:, 1], zpad], axis=0)


_tc_gatpre = pl.pallas_call(
    _tc_gatpre_body,
    out_shape=(
        jax.ShapeDtypeStruct((N, DF), jnp.float32),
        jax.ShapeDtypeStruct((N, DF), jnp.float32),
        jax.ShapeDtypeStruct((NP,), jnp.float32),
        jax.ShapeDtypeStruct((NP,), jnp.float32),
    ),
)


def _tc_post_body(acc_ref, den_ref, es_ref, ed_ref, hwg_ref, h3_ref,
                  bg_ref, ga_ref, bea_ref, batch_ref,
                  wm1_ref, bm1_ref, wh_ref, bh_ref, wm2_ref, bm2_ref,
                  temp_ref, z_ref):
    es = es_ref[:N]
    ed = ed_ref[:N]
    e_self = es + ed
    e_self = jnp.where(e_self >= 0.0, e_self, e_self * jnp.float32(0.2))
    ee_self = jnp.exp(e_self)
    hwg = hwg_ref[...]
    numer = acc_ref[0, :N, :] + acc_ref[1, :N, :] + ee_self[:, None] * hwg
    den = den_ref[0, :N] + den_ref[1, :N] + ee_self
    gat = numer / den[:, None] + bg_ref[...]
    h = jax.nn.relu(_bn(gat, ga_ref[...], bea_ref[...])) + h3_ref[...]
    onehot = (batch_ref[...][:, None]
              == lax.broadcasted_iota(jnp.int32, (N, NG), 1))
    onehot = onehot.astype(jnp.float32)
    sums = lax.dot_general(onehot, h, (((0,), (0,)), ((), ())))
    cnt = jnp.sum(onehot, axis=0)
    pooled = sums / jnp.maximum(cnt, 1.0)[:, None]
    z = jax.nn.relu(pooled @ wm1_ref[...] + bm1_ref[...])
    z = jax.nn.relu(z @ wh_ref[...] + bh_ref[...])
    z = z @ wm2_ref[...] + bm2_ref[...]
    z_ref[...] = z * jnp.exp(temp_ref[0])


_tc_post = pl.pallas_call(
    _tc_post_body,
    out_shape=jax.ShapeDtypeStruct((NG, 128), jnp.float32),
)


def kernel(x, edge_index, batch, W1, b1, W2, b2, W3, b3, g1, be1, g2, be2,
           g3, be3, Wg, a_s, a_d, bg, ga, bea, Wm1, bm1, Wh, bh, Wm2, bm2,
           temp):
    e = edge_index.shape[1]
    quant = NW * CH
    e_pad = -(-e // quant) * quant
    pad = e_pad - e
    epw = e_pad // NW
    epc = epw // CH
    src_p = jnp.concatenate(
        [edge_index[0], jnp.zeros((pad,), edge_index.dtype)])
    dst_p = jnp.concatenate(
        [edge_index[1], jnp.full((pad,), N, edge_index.dtype)])
    dst_p3 = dst_p.reshape(NW, epc, CH)

    deg_k = _make_deg_kernel(epc)
    gcn_k = _make_gcn_kernel(epc)
    gat_k = _make_gat_kernel(epc)

    deg2 = deg_k(dst_p3)
    hws1, dinv = _tc_pre(x, W1, deg2)
    acc1 = gcn_k(src_p, dst_p, hws1)
    h1, hws2 = _tc_mid(acc1, hws1, dinv, b1, g1, be1, x, W2)
    acc2 = gcn_k(src_p, dst_p, hws2)
    h2, hws3 = _tc_mid(acc2, hws2, dinv, b2, g2, be2, h1, W3)
    acc3 = gcn_k(src_p, dst_p, hws3)
    h3, hwg, es_p, ed_p = _tc_gatpre(acc3, hws3, dinv, b3, g3, be3, h2,
                                     Wg, a_s, a_d)
    accg, deng = gat_k(src_p, dst_p, hwg, es_p, ed_p)
    z = _tc_post(accg, deng, es_p, ed_p, hwg, h3, bg, ga, bea, batch,
                 Wm1, bm1, Wh, bh, Wm2, bm2, temp)
    return z
